# Initial kernel scaffold; baseline (speedup 1.0000x reference)
import jax
import jax.numpy as jnp
from jax.experimental import pallas as pl


def kernel(x, edge_index, W1, b1, W2, b2, s1_Wl, s1_Wr, s1_b, s2_Wl, s2_Wr, s2_b):
    raise NotImplementedError("write your pallas kernel here")


# speedup vs baseline: 10.2431x; 10.2431x over previous
import jax
import jax.numpy as jnp
from jax import lax
from jax.experimental import pallas as pl
from jax.experimental.pallas import tpu as pltpu
from jax.experimental.pallas import tpu_sc as plsc

N = 10000
NPAD = 10240
E = 320000
EPAD = 327680
NSLICE = 8
CH = 128
RB = 400
GRID = N // RB

NC, NS = 2, 16
ROWS_PER_TILE = NPAD // NS
NCHG = EPAD // CH
NCH1 = NCHG // NS
NCH2 = NCHG // (NC * NS)


def _mlp_body(x_ref, w1_ref, b1_ref, w2_ref, b2_ref, h2_ref):
    h = jnp.maximum(
        jnp.dot(x_ref[...], w1_ref[...], preferred_element_type=jnp.float32)
        + b1_ref[0, :][None, :], 0.0)
    h2_ref[...] = jnp.maximum(
        jnp.dot(h, w2_ref[...], preferred_element_type=jnp.float32)
        + b2_ref[0, :][None, :], 0.0)


def _mlp(x, W1, b1, W2, b2):
    return pl.pallas_call(
        _mlp_body,
        grid=(GRID,),
        in_specs=[
            pl.BlockSpec((RB, 128), lambda i: (i, 0)),
            pl.BlockSpec((128, 4096), lambda i: (0, 0)),
            pl.BlockSpec((1, 4096), lambda i: (0, 0)),
            pl.BlockSpec((4096, 2048), lambda i: (0, 0)),
            pl.BlockSpec((1, 2048), lambda i: (0, 0)),
        ],
        out_specs=pl.BlockSpec((RB, 2048), lambda i: (i, 0)),
        out_shape=jax.ShapeDtypeStruct((N, 2048), jnp.float32),
        compiler_params=pltpu.CompilerParams(
            dimension_semantics=("arbitrary",)),
    )(x, W1, b1, W2, b2)


def _proj1_body(h2_ref, wl_ref, wr_ref, sb_ref, p1_ref, r1_ref):
    h = h2_ref[...]
    p1 = jnp.dot(h, wl_ref[...], preferred_element_type=jnp.float32)
    r1_ref[...] = (
        jnp.dot(h, wr_ref[...], preferred_element_type=jnp.float32)
        + sb_ref[0, :][None, :])
    for f in range(NSLICE):
        p1_ref[f] = p1[:, 128 * f:128 * (f + 1)]


def _proj1(h2, Wl, Wr, sb):
    return pl.pallas_call(
        _proj1_body,
        grid=(GRID,),
        in_specs=[
            pl.BlockSpec((RB, 2048), lambda i: (i, 0)),
            pl.BlockSpec((2048, 1024), lambda i: (0, 0)),
            pl.BlockSpec((2048, 1024), lambda i: (0, 0)),
            pl.BlockSpec((1, 1024), lambda i: (0, 0)),
        ],
        out_specs=[
            pl.BlockSpec((NSLICE, RB, 128), lambda i: (0, i, 0)),
            pl.BlockSpec((RB, 1024), lambda i: (i, 0)),
        ],
        out_shape=[
            jax.ShapeDtypeStruct((NSLICE, N, 128), jnp.float32),
            jax.ShapeDtypeStruct((N, 1024), jnp.float32),
        ],
        compiler_params=pltpu.CompilerParams(
            dimension_semantics=("arbitrary",)),
    )(h2, Wl, Wr, sb)


def _seg1_body(p1_hbm, sd_hbm, agg_hbm, cnt_hbm,
               i0, i1, buf0, buf1, ones, zbuf, zcnt, accum, cnt_acc,
               semi0, semi1, semg0, semg1):
    cid = lax.axis_index("c")
    sid = lax.axis_index("s")
    zero16 = jnp.zeros((16,), jnp.float32)
    one16 = jnp.ones((16,), jnp.float32)

    def zrow(i, c):
        for j in range(8):
            zbuf[i, pl.ds(j * 16, 16)] = zero16
        return c
    lax.fori_loop(0, 64, zrow, 0)
    for j in range(CH // 16):
        ones[pl.ds(j * 16, 16)] = one16

    def zc(i, c):
        zcnt[pl.ds(i * 16, 16)] = zero16
        return c
    lax.fori_loop(0, ROWS_PER_TILE // 16, zc, 0)
    pltpu.sync_copy(zcnt, cnt_acc.at[pl.ds(sid * ROWS_PER_TILE,
                                           ROWS_PER_TILE)])

    for k in range(NSLICE // NC):
        f = cid * (NSLICE // NC) + k
        for r in range(ROWS_PER_TILE // 64):
            pltpu.sync_copy(
                zbuf, accum.at[pl.ds(sid * ROWS_PER_TILE + r * 64, 64)])
        plsc.subcore_barrier()

        rb = f * NCHG + sid * NCH1

        pltpu.async_copy(sd_hbm.at[rb], i0, semi0)
        pltpu.async_copy(sd_hbm.at[rb + 1], i1, semi1)

        def pair(io, c):
            c0 = rb + 2 * io
            pltpu.make_async_copy(sd_hbm.at[c0], i0, semi0).wait()
            pltpu.async_copy(p1_hbm.at[i0.at[0]], buf0, semg0)
            pltpu.make_async_copy(sd_hbm.at[c0 + 1], i1, semi1).wait()
            pltpu.async_copy(p1_hbm.at[i1.at[0]], buf1, semg1)

            pltpu.make_async_copy(p1_hbm.at[i0.at[0]], buf0, semg0).wait()
            pltpu.sync_copy(buf0, accum.at[i0.at[1]], add=True)
            if k == 0:
                pltpu.sync_copy(ones, cnt_acc.at[i0.at[1]], add=True)

            @pl.when(io < NCH1 // 2 - 1)
            def _():
                pltpu.async_copy(sd_hbm.at[c0 + 2], i0, semi0)

            pltpu.make_async_copy(p1_hbm.at[i1.at[0]], buf1, semg1).wait()
            pltpu.sync_copy(buf1, accum.at[i1.at[1]], add=True)
            if k == 0:
                pltpu.sync_copy(ones, cnt_acc.at[i1.at[1]], add=True)

            @pl.when(io < NCH1 // 2 - 1)
            def _():
                pltpu.async_copy(sd_hbm.at[c0 + 3], i1, semi1)
            return c
        lax.fori_loop(0, NCH1 // 2, pair, 0)
        plsc.subcore_barrier()
        pltpu.sync_copy(
            accum.at[pl.ds(sid * ROWS_PER_TILE, ROWS_PER_TILE)],
            agg_hbm.at[pl.ds(f * NPAD + sid * ROWS_PER_TILE,
                             ROWS_PER_TILE)])
    pltpu.sync_copy(
        cnt_acc.at[pl.ds(sid * ROWS_PER_TILE, ROWS_PER_TILE)],
        cnt_hbm.at[pl.ds(cid * NPAD + sid * ROWS_PER_TILE, ROWS_PER_TILE)])


def _seg1(p1_flat, sd):
    mesh = plsc.VectorSubcoreMesh(core_axis_name="c", subcore_axis_name="s",
                                  num_cores=NC, num_subcores=NS)
    return pl.kernel(
        _seg1_body,
        out_type=[
            jax.ShapeDtypeStruct((NSLICE * NPAD, 128), jnp.float32),
            jax.ShapeDtypeStruct((NC * NPAD,), jnp.float32),
        ],
        mesh=mesh,
        scratch_types=[
            pltpu.VMEM((2, CH), jnp.int32),
            pltpu.VMEM((2, CH), jnp.int32),
            pltpu.VMEM((CH, 128), jnp.float32),
            pltpu.VMEM((CH, 128), jnp.float32),
            pltpu.VMEM((CH,), jnp.float32),
            pltpu.VMEM((64, 128), jnp.float32),
            pltpu.VMEM((ROWS_PER_TILE,), jnp.float32),
            pltpu.VMEM_SHARED((NPAD, 128), jnp.float32),
            pltpu.VMEM_SHARED((NPAD,), jnp.float32),
            pltpu.SemaphoreType.DMA,
            pltpu.SemaphoreType.DMA,
            pltpu.SemaphoreType.DMA,
            pltpu.SemaphoreType.DMA,
        ],
    )(p1_flat, sd)


def _combine1_body(agg_ref, cnt_ref, r1_ref, wl_ref, wr_ref, sb_ref,
                   p2_ref, r2_ref):
    recip = 1.0 / jnp.maximum(cnt_ref[0, 0, :], 1.0)
    p2 = jnp.zeros((RB, 64), jnp.float32)
    r2 = jnp.zeros((RB, 64), jnp.float32)
    for f in range(NSLICE):
        g = jnp.maximum(
            agg_ref[f] * recip[:, None] + r1_ref[:, 128 * f:128 * (f + 1)],
            0.0)
        p2 = p2 + jnp.dot(g, wl_ref[128 * f:128 * (f + 1), :],
                          preferred_element_type=jnp.float32)
        r2 = r2 + jnp.dot(g, wr_ref[128 * f:128 * (f + 1), :],
                          preferred_element_type=jnp.float32)
    p2_ref[:, 0:64] = p2
    p2_ref[:, 64:128] = jnp.zeros((RB, 64), jnp.float32)
    r2_ref[...] = r2 + sb_ref[0, :][None, :]


def _combine1(agg1, cnt, r1, Wl, Wr, sb):
    return pl.pallas_call(
        _combine1_body,
        grid=(GRID,),
        in_specs=[
            pl.BlockSpec((NSLICE, RB, 128), lambda i: (0, i, 0)),
            pl.BlockSpec((1, 1, RB), lambda i: (i, 0, 0)),
            pl.BlockSpec((RB, 1024), lambda i: (i, 0)),
            pl.BlockSpec((1024, 64), lambda i: (0, 0)),
            pl.BlockSpec((1024, 64), lambda i: (0, 0)),
            pl.BlockSpec((1, 64), lambda i: (0, 0)),
        ],
        out_specs=[
            pl.BlockSpec((RB, 128), lambda i: (i, 0)),
            pl.BlockSpec((RB, 64), lambda i: (i, 0)),
        ],
        out_shape=[
            jax.ShapeDtypeStruct((N, 128), jnp.float32),
            jax.ShapeDtypeStruct((N, 64), jnp.float32),
        ],
        compiler_params=pltpu.CompilerParams(
            dimension_semantics=("arbitrary",)),
    )(agg1, cnt, r1, Wl, Wr, sb)


def _seg2_body(p2_hbm, sd_hbm, agg_hbm,
               i0, i1, buf0, buf1, zbuf, accum, semi0, semi1, semg0, semg1):
    cid = lax.axis_index("c")
    sid = lax.axis_index("s")
    zero16 = jnp.zeros((16,), jnp.float32)

    def zrow(i, c):
        for j in range(8):
            zbuf[i, pl.ds(j * 16, 16)] = zero16
        return c
    lax.fori_loop(0, 64, zrow, 0)
    for r in range(ROWS_PER_TILE // 64):
        pltpu.sync_copy(
            zbuf, accum.at[pl.ds(sid * ROWS_PER_TILE + r * 64, 64)])
    plsc.subcore_barrier()

    rb = (cid * NS + sid) * NCH2
    pltpu.async_copy(sd_hbm.at[rb], i0, semi0)
    pltpu.async_copy(sd_hbm.at[rb + 1], i1, semi1)

    def pair(io, c):
        c0 = rb + 2 * io
        pltpu.make_async_copy(sd_hbm.at[c0], i0, semi0).wait()
        pltpu.async_copy(p2_hbm.at[i0.at[0]], buf0, semg0)
        pltpu.make_async_copy(sd_hbm.at[c0 + 1], i1, semi1).wait()
        pltpu.async_copy(p2_hbm.at[i1.at[0]], buf1, semg1)

        pltpu.make_async_copy(p2_hbm.at[i0.at[0]], buf0, semg0).wait()
        pltpu.sync_copy(buf0, accum.at[i0.at[1]], add=True)

        @pl.when(io < NCH2 // 2 - 1)
        def _():
            pltpu.async_copy(sd_hbm.at[c0 + 2], i0, semi0)

        pltpu.make_async_copy(p2_hbm.at[i1.at[0]], buf1, semg1).wait()
        pltpu.sync_copy(buf1, accum.at[i1.at[1]], add=True)

        @pl.when(io < NCH2 // 2 - 1)
        def _():
            pltpu.async_copy(sd_hbm.at[c0 + 3], i1, semi1)
        return c
    lax.fori_loop(0, NCH2 // 2, pair, 0)
    plsc.subcore_barrier()
    pltpu.sync_copy(
        accum.at[pl.ds(sid * ROWS_PER_TILE, ROWS_PER_TILE)],
        agg_hbm.at[pl.ds(cid * NPAD + sid * ROWS_PER_TILE, ROWS_PER_TILE)])


def _seg2(p2, sd):
    mesh = plsc.VectorSubcoreMesh(core_axis_name="c", subcore_axis_name="s",
                                  num_cores=NC, num_subcores=NS)
    return pl.kernel(
        _seg2_body,
        out_type=jax.ShapeDtypeStruct((NC * NPAD, 128), jnp.float32),
        mesh=mesh,
        scratch_types=[
            pltpu.VMEM((2, CH), jnp.int32),
            pltpu.VMEM((2, CH), jnp.int32),
            pltpu.VMEM((CH, 128), jnp.float32),
            pltpu.VMEM((CH, 128), jnp.float32),
            pltpu.VMEM((64, 128), jnp.float32),
            pltpu.VMEM_SHARED((NPAD, 128), jnp.float32),
            pltpu.SemaphoreType.DMA,
            pltpu.SemaphoreType.DMA,
            pltpu.SemaphoreType.DMA,
            pltpu.SemaphoreType.DMA,
        ],
    )(p2, sd)


def _final_body(agg_ref, cnt_ref, r2_ref, out_ref):
    recip = 1.0 / jnp.maximum(cnt_ref[0, 0, :], 1.0)
    z = jnp.maximum(
        (agg_ref[0, :, 0:64] + agg_ref[1, :, 0:64]) * recip[:, None]
        + r2_ref[...], 0.0)
    z = z - jnp.max(z, axis=1, keepdims=True)
    ez = jnp.exp(z)
    out_ref[...] = ez / jnp.sum(ez, axis=1, keepdims=True)


def _final(agg2, cnt, r2):
    return pl.pallas_call(
        _final_body,
        grid=(GRID,),
        in_specs=[
            pl.BlockSpec((NC, RB, 128), lambda i: (0, i, 0)),
            pl.BlockSpec((1, 1, RB), lambda i: (i, 0, 0)),
            pl.BlockSpec((RB, 64), lambda i: (i, 0)),
        ],
        out_specs=pl.BlockSpec((RB, 64), lambda i: (i, 0)),
        out_shape=jax.ShapeDtypeStruct((N, 64), jnp.float32),
        compiler_params=pltpu.CompilerParams(
            dimension_semantics=("arbitrary",)),
    )(agg2, cnt, r2)


def kernel(x, edge_index, W1, b1, W2, b2, s1_Wl, s1_Wr, s1_b,
           s2_Wl, s2_Wr, s2_b):
    npad = EPAD - E
    pad_src = jnp.arange(npad, dtype=edge_index.dtype) % N
    pad_dst = N + jnp.arange(npad, dtype=edge_index.dtype) % (NPAD - N)
    src = jnp.concatenate([edge_index[0], pad_src]).reshape(NCHG, CH)
    dst = jnp.concatenate([edge_index[1], pad_dst]).reshape(NCHG, CH)
    offs = (jnp.arange(NSLICE, dtype=src.dtype) * N)[:, None, None]
    src8 = src[None, :, :] + offs
    dst8 = jnp.broadcast_to(dst[None, :, :], (NSLICE, NCHG, CH))
    sd = jnp.stack([src8, dst8], axis=2).reshape(NSLICE * NCHG, 2, CH)

    h2 = _mlp(x, W1, b1.reshape(1, 4096), W2, b2.reshape(1, 2048))
    p1s, r1 = _proj1(h2, s1_Wl, s1_Wr, s1_b.reshape(1, 1024))
    agg1_flat, cnt_flat = _seg1(p1s.reshape(NSLICE * N, 128), sd)
    agg1 = agg1_flat.reshape(NSLICE, NPAD, 128)
    cnt = cnt_flat[:N].reshape(GRID, 1, RB)
    p2, r2 = _combine1(agg1, cnt, r1, s2_Wl, s2_Wr, s2_b.reshape(1, 64))
    agg2 = _seg2(p2, sd[:NCHG]).reshape(NC, NPAD, 128)
    return _final(agg2, cnt, r2)
